# baseline (speedup 1.0000x reference)
import functools

import jax
import jax.numpy as jnp
from jax import lax
from jax.experimental import pallas as pl
from jax.experimental.pallas import tpu as pltpu
from jax.experimental.pallas import tpu_sc as plsc

VOC = 1000000
EMB = 64
B = 4096
T = 20

NC = 2
NS = 16
NW = NC * NS
BPW = B // NW
CB = 4
ROWS = CB * T
NCHUNK = BPW // CB
LANES = 16
EC = EMB // LANES


def _sc_logits(inpt, trgs_flat, rand_flat, emb_table, lin_w):
  mesh = plsc.VectorSubcoreMesh(core_axis_name="c", subcore_axis_name="s")

  @functools.partial(
      pl.kernel,
      out_type=[
          jax.ShapeDtypeStruct((B * T,), jnp.float32),
          jax.ShapeDtypeStruct((B * T,), jnp.float32),
      ],
      mesh=mesh,
      compiler_params=pltpu.CompilerParams(use_tc_tiling_on_sc=False),
      scratch_types=[
          pltpu.VMEM((BPW,), jnp.int32),
          pltpu.VMEM((BPW, EMB), jnp.float32),
          pltpu.VMEM((BPW * T,), jnp.int32),
          pltpu.VMEM((BPW * T,), jnp.int32),
          pltpu.VMEM((2, ROWS, EMB), jnp.float32),
          pltpu.VMEM((BPW * T,), jnp.float32),
          pltpu.VMEM((BPW * T,), jnp.float32),
          pltpu.SemaphoreType.DMA,
          pltpu.SemaphoreType.DMA,
          pltpu.SemaphoreType.DMA,
      ],
  )
  def k(inpt_h, trgs_h, rand_h, emb_h, lin_h, pos_h, neg_h,
        iidx, embv, tidx, ridx, rows2, posv, negv, sem0, sem1, sem_e):
    wid = lax.axis_index("s") * NC + lax.axis_index("c")
    base = wid * BPW

    pltpu.sync_copy(inpt_h.at[pl.ds(base, BPW)], iidx)
    emb_cp = pltpu.async_copy(emb_h.at[iidx], embv, sem_e)
    pltpu.sync_copy(trgs_h.at[pl.ds(base * T, BPW * T)], tidx)
    pltpu.sync_copy(rand_h.at[pl.ds(base * T, BPW * T)], ridx)
    emb_cp.wait()

    sems = (sem0, sem1)

    def start(idxref, g, slot):
      pltpu.async_copy(
          lin_h.at[idxref.at[pl.ds(g * ROWS, ROWS)]],
          rows2.at[slot], sems[slot])

    def wait(idxref, g, slot):
      pltpu.make_async_copy(
          lin_h.at[idxref.at[pl.ds(g * ROWS, ROWS)]],
          rows2.at[slot], sems[slot]).wait()

    lane_masks = [lax.iota(jnp.int32, LANES) == j for j in range(LANES)]
    perms = [lax.iota(jnp.int32, LANES) ^ sh for sh in (8, 4, 2, 1)]

    def compute(outflat, g, slot):
      res = jnp.zeros((LANES,), jnp.float32)
      e = None
      for d in range(ROWS):
        cb, t = divmod(d, T)
        if t == 0:
          e = [embv[g * CB + cb, pl.ds(c * LANES, LANES)] for c in range(EC)]
        acc = rows2[slot, d, pl.ds(0, LANES)] * e[0]
        for c in range(1, EC):
          acc = acc + rows2[slot, d, pl.ds(c * LANES, LANES)] * e[c]
        for p in perms:
          acc = acc + jnp.take(acc, p)
        res = jnp.where(lane_masks[d % LANES], acc, res)
        if d % LANES == LANES - 1:
          outflat[pl.ds(g * ROWS + (d - LANES + 1), LANES)] = res

    def run_table(idxref, outref):
      start(idxref, 0, 0)

      def body(i, carry):
        g0 = 2 * i

        @pl.when(g0 + 1 < NCHUNK)
        def _():
          start(idxref, g0 + 1, 1)

        wait(idxref, g0, 0)
        compute(outref, g0, 0)

        @pl.when(g0 + 2 < NCHUNK)
        def _():
          start(idxref, g0 + 2, 0)

        @pl.when(g0 + 1 < NCHUNK)
        def _():
          wait(idxref, g0 + 1, 1)
          compute(outref, g0 + 1, 1)

        return carry

      lax.fori_loop(0, NCHUNK // 2, body, 0)

    run_table(tidx, posv)
    run_table(ridx, negv)

    pltpu.sync_copy(posv, pos_h.at[pl.ds(base * T, BPW * T)])
    pltpu.sync_copy(negv, neg_h.at[pl.ds(base * T, BPW * T)])

  return k(inpt, trgs_flat, rand_flat, emb_table, lin_w)


def _tc_loss(pos, neg):
  def body(pos_ref, neg_ref, o_ref):
    p = jax.nn.sigmoid(pos_ref[...])
    n = jax.nn.sigmoid(neg_ref[...])
    pst = -jnp.mean(jnp.log(p))
    ngt = -jnp.mean(jnp.log(1.0 - n + 1e-3))
    o_ref[0, 0] = pst + ngt

  out = pl.pallas_call(
      body,
      out_shape=jax.ShapeDtypeStruct((1, 1), jnp.float32),
      in_specs=[
          pl.BlockSpec(memory_space=pltpu.VMEM),
          pl.BlockSpec(memory_space=pltpu.VMEM),
      ],
      out_specs=pl.BlockSpec(memory_space=pltpu.SMEM),
  )(pos, neg)
  return out[0, 0]


def kernel(inpt, trgs, rand, emb_table, lin_w):
  inpt = inpt.astype(jnp.int32)
  trgs_flat = trgs.astype(jnp.int32).reshape(-1)
  rand_flat = rand.astype(jnp.int32).reshape(-1)
  pos, neg = _sc_logits(inpt, trgs_flat, rand_flat, emb_table, lin_w)
  return _tc_loss(pos.reshape(B * T // 128, 128), neg.reshape(B * T // 128, 128))
